# baseline (speedup 1.0000x reference)
import functools

import jax
import jax.numpy as jnp
from jax import lax
from jax.experimental import pallas as pl
from jax.experimental.pallas import tpu as pltpu, tpu_sc as plsc


def kernel(token_embedding, pos_table):
    B, S, E = token_embedding.shape
    NC, NS = 2, 16
    NW = NC * NS
    total = B * S * E
    per_w = total // NW
    RCH = 16
    C = RCH * E
    nch = per_w // C
    pos_elems = S * E

    te_flat = token_embedding.reshape(-1)
    pos_flat = pos_table[:S].reshape(-1)

    mesh = plsc.VectorSubcoreMesh(core_axis_name="c", subcore_axis_name="s")

    @functools.partial(
        pl.kernel,
        out_type=jax.ShapeDtypeStruct((total,), jnp.float32),
        mesh=mesh,
        scratch_types=[
            pltpu.VMEM((C,), jnp.float32),
            pltpu.VMEM((C,), jnp.float32),
        ],
    )
    def sc_add(te_hbm, pos_hbm, out_hbm, te_v, pos_v):
        wid = lax.axis_index("s") * NC + lax.axis_index("c")
        base = wid * per_w

        def chunk_body(g, carry):
            off = pl.multiple_of(base + g * C, C)
            poff = pl.multiple_of(lax.rem(off, pos_elems), C)
            pltpu.sync_copy(te_hbm.at[pl.ds(off, C)], te_v)
            pltpu.sync_copy(pos_hbm.at[pl.ds(poff, C)], pos_v)

            def add_body(i, c2):
                sl = pl.ds(i * 16, 16)
                te_v[sl] = te_v[sl] + pos_v[sl]
                return c2

            lax.fori_loop(0, C // 16, add_body, 0)
            pltpu.sync_copy(te_v, out_hbm.at[pl.ds(off, C)])
            return carry

        lax.fori_loop(0, nch, chunk_body, 0)

    out = sc_add(te_flat, pos_flat)
    return out.reshape(B, S, E)


# speedup vs baseline: 9.0330x; 9.0330x over previous
import functools

import jax
import jax.numpy as jnp
from jax.experimental import pallas as pl


def _add_kernel(te_ref, pos_ref, out_ref, *, bs):
    i = pl.program_id(0)
    out_ref[0] = te_ref[0] + pos_ref[pl.ds(i * bs, bs), :]


def kernel(token_embedding, pos_table):
    B, S, E = token_embedding.shape
    BS = 2048
    grid = (S // BS, B)
    return pl.pallas_call(
        functools.partial(_add_kernel, bs=BS),
        grid=grid,
        in_specs=[
            pl.BlockSpec((1, BS, E), lambda i, j: (j, i, 0)),
            pl.BlockSpec((S, E), lambda i, j: (0, 0)),
        ],
        out_specs=pl.BlockSpec((1, BS, E), lambda i, j: (j, i, 0)),
        out_shape=jax.ShapeDtypeStruct((B, S, E), token_embedding.dtype),
    )(token_embedding, pos_table)


# speedup vs baseline: 9.0772x; 1.0049x over previous
import functools

import jax
import jax.numpy as jnp
from jax import lax
from jax.experimental import pallas as pl
from jax.experimental.pallas import tpu as pltpu

NB = 4
CH = 512


def _pipeline_kernel(te_hbm, pos_hbm, out_hbm, te_bufs, out_bufs, pos_buf,
                     lsem, ssem, psem, *, B, S, E):
    cpb = S // CH
    total = B * cpb

    def load_te(c, slot):
        b = c // cpb
        r = c - b * cpb
        off = pl.multiple_of(r * CH, CH)
        return pltpu.make_async_copy(
            te_hbm.at[b, pl.ds(off, CH)], te_bufs.at[slot], lsem.at[slot])

    def load_pos(c, slot):
        off = pl.multiple_of(c * CH, CH)
        return pltpu.make_async_copy(
            pos_hbm.at[pl.ds(off, CH)], pos_buf.at[pl.ds(off, CH)],
            psem.at[slot])

    def store_out(c, slot):
        b = c // cpb
        r = c - b * cpb
        off = pl.multiple_of(r * CH, CH)
        return pltpu.make_async_copy(
            out_bufs.at[slot], out_hbm.at[b, pl.ds(off, CH)], ssem.at[slot])

    for slot in range(NB):
        load_te(slot, slot).start()
        load_pos(slot, slot).start()

    def group(g, carry):
        for slot in range(NB):
            c = g * NB + slot
            load_te(c, slot).wait()

            @pl.when(c < cpb)
            def _():
                load_pos(c, slot).wait()

            @pl.when(c >= NB)
            def _():
                store_out(c - NB, slot).wait()

            r = c - (c // cpb) * cpb
            off = pl.multiple_of(r * CH, CH)
            out_bufs[slot] = te_bufs[slot] + pos_buf[pl.ds(off, CH), :]
            store_out(c, slot).start()

            nxt = c + NB

            @pl.when(nxt < total)
            def _():
                load_te(nxt, slot).start()

            @pl.when(nxt < cpb)
            def _():
                load_pos(nxt, slot).start()
        return carry

    lax.fori_loop(0, total // NB, group, 0)

    for slot in range(NB):
        pltpu.make_async_copy(
            out_bufs.at[slot], out_hbm.at[0, pl.ds(0, CH)],
            ssem.at[slot]).wait()


def kernel(token_embedding, pos_table):
    B, S, E = token_embedding.shape
    return pl.pallas_call(
        functools.partial(_pipeline_kernel, B=B, S=S, E=E),
        in_specs=[
            pl.BlockSpec(memory_space=pl.ANY),
            pl.BlockSpec(memory_space=pl.ANY),
        ],
        out_specs=pl.BlockSpec(memory_space=pl.ANY),
        out_shape=jax.ShapeDtypeStruct((B, S, E), token_embedding.dtype),
        scratch_shapes=[
            pltpu.VMEM((NB, CH, E), jnp.float32),
            pltpu.VMEM((NB, CH, E), jnp.float32),
            pltpu.VMEM((S, E), jnp.float32),
            pltpu.SemaphoreType.DMA((NB,)),
            pltpu.SemaphoreType.DMA((NB,)),
            pltpu.SemaphoreType.DMA((NB,)),
        ],
    )(token_embedding, pos_table)
